# baseline (speedup 1.0000x reference)
import jax
import jax.numpy as jnp
from jax import lax
from jax.experimental import pallas as pl
from jax.experimental.pallas import tpu as pltpu
from jax.experimental.pallas import tpu_sc as plsc

N = 10000
D = 128
E = 320000

NC = 2
NS = 16
LANES = 16

CHUNK = 128
NW = NC * NS
G = 8
CPT = ((E + CHUNK * NW * G - 1) // (CHUNK * NW * G)) * G
NCHUNKS = CPT * NW
E_PAD = NCHUNKS * CHUNK
NG = CPT // G
NG2 = NG // 2

NP = 10112
RPT = NP // NS
NF = 4
FW = D // NF
ZB = RPT // 2

_mesh = plsc.VectorSubcoreMesh(core_axis_name="c", subcore_axis_name="s")
_sc_params = pltpu.CompilerParams(use_tc_tiling_on_sc=False)


def _zero_rows(ref, nrows, width):
    z = jnp.zeros((LANES,), jnp.float32)

    def body(i, _):
        for t in range(width // LANES):
            ref[i, pl.ds(t * LANES, LANES)] = z
        return 0

    lax.fori_loop(0, nrows, body, 0, unroll=False)


def _sc_deg_body(srcc, dstc, out_s, out_d, sidx_all, didx_all, ones_v, stage,
                 sh_s, sh_d, sem_s):
    c = lax.axis_index("c")
    s = lax.axis_index("s")
    wid = c * NS + s

    pltpu.sync_copy(srcc.at[pl.ds(wid * CPT, CPT)], sidx_all)
    pltpu.sync_copy(dstc.at[pl.ds(wid * CPT, CPT)], didx_all)

    one = jnp.ones((LANES,), jnp.float32)

    def initones(i, _):
        ones_v[i, :] = one
        return 0

    lax.fori_loop(0, CHUNK, initones, 0, unroll=False)
    _zero_rows(stage, RPT, LANES)

    row0 = s * RPT
    pltpu.sync_copy(stage, sh_s.at[pl.ds(row0, RPT)])
    pltpu.sync_copy(stage, sh_d.at[pl.ds(row0, RPT)])
    plsc.subcore_barrier()

    def group_body(g, _):
        @pl.when(g > 0)
        def _():
            for _b in range(G):
                pltpu.make_async_copy(ones_v, sh_s.at[sidx_all.at[0]], sem_s).wait()
                pltpu.make_async_copy(ones_v, sh_d.at[didx_all.at[0]], sem_s).wait()

        for b in range(G):
            j = g * G + b
            pltpu.async_copy(ones_v, sh_s.at[sidx_all.at[j]], sem_s, add=True)
            pltpu.async_copy(ones_v, sh_d.at[didx_all.at[j]], sem_s, add=True)
        return 0

    lax.fori_loop(0, NG, group_body, 0, unroll=False)
    for _b in range(G):
        pltpu.make_async_copy(ones_v, sh_s.at[sidx_all.at[0]], sem_s).wait()
        pltpu.make_async_copy(ones_v, sh_d.at[didx_all.at[0]], sem_s).wait()
    plsc.subcore_barrier()

    pltpu.sync_copy(sh_s.at[pl.ds(row0, RPT)], stage)
    pltpu.sync_copy(stage, out_s.at[c, pl.ds(row0, RPT)])
    pltpu.sync_copy(sh_d.at[pl.ds(row0, RPT)], stage)
    pltpu.sync_copy(stage, out_d.at[c, pl.ds(row0, RPT)])


def _sc_agg_body(hg, srcc, dstc, out, sidx_all, didx_all, rows_g, zcbuf,
                 sh_acc, sem_g, sem_s):
    c = lax.axis_index("c")
    s = lax.axis_index("s")
    wid = c * NS + s
    row0 = s * RPT

    pltpu.sync_copy(srcc.at[pl.ds(wid * CPT, CPT)], sidx_all)
    pltpu.sync_copy(dstc.at[pl.ds(wid * CPT, CPT)], didx_all)

    def _fire_g(p, grp, base):
        for b in range(G):
            pltpu.async_copy(
                hg.at[p].at[sidx_all.at[grp * G + b]], rows_g.at[base + b], sem_g)

    def _drain_g(p, grp, base):
        for b in range(G):
            pltpu.make_async_copy(
                hg.at[p].at[sidx_all.at[grp * G + b]], rows_g.at[base + b], sem_g).wait()

    def _fire_s(grp, base):
        for b in range(G):
            pltpu.async_copy(
                rows_g.at[base + b], sh_acc.at[didx_all.at[grp * G + b]], sem_s, add=True)

    def _drain_s(base):
        for b in range(G):
            pltpu.make_async_copy(
                rows_g.at[base + b], sh_acc.at[didx_all.at[0]], sem_s).wait()

    for p in range(NF):
        _zero_rows(zcbuf, ZB, FW)
        pltpu.sync_copy(zcbuf, sh_acc.at[pl.ds(row0, ZB)])
        pltpu.sync_copy(zcbuf, sh_acc.at[pl.ds(row0 + ZB, ZB)])
        plsc.subcore_barrier()

        _fire_g(p, 0, 0)

        def pair_body(g2, _):
            @pl.when(g2 > 0)
            def _():
                _drain_s(G)
            _fire_g(p, 2 * g2 + 1, G)
            _drain_g(p, 2 * g2, 0)
            _fire_s(2 * g2, 0)
            _drain_g(p, 2 * g2 + 1, G)
            _fire_s(2 * g2 + 1, G)

            @pl.when(g2 < NG2 - 1)
            def _():
                _drain_s(0)
                _fire_g(p, 2 * g2 + 2, 0)
            return 0

        lax.fori_loop(0, NG2, pair_body, 0, unroll=False)
        _drain_s(0)
        _drain_s(G)
        plsc.subcore_barrier()

        for hb in range(2):
            r = row0 + hb * ZB
            pltpu.sync_copy(sh_acc.at[pl.ds(r, ZB)], zcbuf)
            pltpu.sync_copy(zcbuf, out.at[c, pl.ds(r, ZB), pl.ds(p * FW, FW)])


_sc_deg = jax.jit(pl.kernel(
    _sc_deg_body,
    out_type=(
        jax.ShapeDtypeStruct((NC, NP, LANES), jnp.float32),
        jax.ShapeDtypeStruct((NC, NP, LANES), jnp.float32),
    ),
    mesh=_mesh,
    compiler_params=_sc_params,
    scratch_types=[
        pltpu.VMEM((CPT, CHUNK), jnp.int32),
        pltpu.VMEM((CPT, CHUNK), jnp.int32),
        pltpu.VMEM((CHUNK, LANES), jnp.float32),
        pltpu.VMEM((RPT, LANES), jnp.float32),
        pltpu.VMEM_SHARED((NP, LANES), jnp.float32),
        pltpu.VMEM_SHARED((NP, LANES), jnp.float32),
        pltpu.SemaphoreType.DMA,
    ],
))

_sc_agg = jax.jit(pl.kernel(
    _sc_agg_body,
    out_type=jax.ShapeDtypeStruct((NC, NP, D), jnp.float32),
    mesh=_mesh,
    compiler_params=_sc_params,
    scratch_types=[
        pltpu.VMEM((CPT, CHUNK), jnp.int32),
        pltpu.VMEM((CPT, CHUNK), jnp.int32),
        pltpu.VMEM((2 * G, CHUNK, FW), jnp.float32),
        pltpu.VMEM((ZB, FW), jnp.float32),
        pltpu.VMEM_SHARED((NP, FW), jnp.float32),
        pltpu.SemaphoreType.DMA,
        pltpu.SemaphoreType.DMA,
    ],
))


def _leaky(x):
    return jnp.where(x >= 0, x, 0.2 * x)


def _tc_norms_body(ds_ref, dd_ref, nsd_out):
    deg_s = ds_ref[0, :, 0:1] + ds_ref[1, :, 0:1]
    deg_d = dd_ref[0, :, 0:1] + dd_ref[1, :, 0:1]
    nsd_out[:, 0:1] = jnp.where(deg_s > 0, lax.rsqrt(deg_s), 0.0)
    nsd_out[:, 1:2] = jnp.where(deg_d > 0, lax.rsqrt(deg_d), 0.0)


def _tc_pre_body(x_ref, w_ref, nsd_ref, h_out):
    h = jnp.dot(x_ref[...], w_ref[...], preferred_element_type=jnp.float32)
    hs = h * nsd_ref[pl.ds(0, N), 0:1]
    for p in range(NF):
        h_out[p, pl.ds(0, N), :] = hs[:, p * FW:(p + 1) * FW]
        h_out[p, pl.ds(N, NP - N), :] = jnp.zeros((NP - N, FW), jnp.float32)


def _united_norm_leaky(h, lam, gamma, beta):
    eps = 1e-5
    mn = jnp.mean(h, axis=1, keepdims=True)
    vn = jnp.mean((h - mn) ** 2, axis=1, keepdims=True)
    mb = jnp.mean(h, axis=0, keepdims=True)
    vb = jnp.mean((h - mb) ** 2, axis=0, keepdims=True)
    mg = jnp.mean(h)
    vg = jnp.mean((h - mg) ** 2)
    e = jnp.exp(lam - jnp.max(lam))
    sinv = 1.0 / jnp.sum(e)
    w0 = jnp.sum(e[:, 0:1]) * sinv
    w1 = jnp.sum(e[:, 1:2]) * sinv
    w2 = jnp.sum(e[:, 2:3]) * sinv
    rn = lax.rsqrt(vn + eps)
    rb = lax.rsqrt(vb + eps)
    rg = lax.rsqrt(vg + eps)
    scale = w0 * rn + w1 * rb + w2 * rg
    shift = w0 * mn * rn + w1 * mb * rb + w2 * mg * rg
    xh = h * scale - shift
    return _leaky(gamma * xh + beta)


def _tc_mid_body(p_ref, nsd_ref, lam_ref, g_ref, b_ref, w_ref, out_ref):
    agg = p_ref[0, pl.ds(0, N), :] + p_ref[1, pl.ds(0, N), :]
    h = agg * nsd_ref[pl.ds(0, N), 1:2]
    y = _united_norm_leaky(h, lam_ref[...], g_ref[...], b_ref[...])
    hn = jnp.dot(y, w_ref[...], preferred_element_type=jnp.float32)
    hs = hn * nsd_ref[pl.ds(0, N), 0:1]
    for p in range(NF):
        out_ref[p, pl.ds(0, N), :] = hs[:, p * FW:(p + 1) * FW]
        out_ref[p, pl.ds(N, NP - N), :] = jnp.zeros((NP - N, FW), jnp.float32)


def _tc_final_body(p_ref, nsd_ref, lam_ref, g_ref, b_ref, out_ref):
    agg = p_ref[0, pl.ds(0, N), :] + p_ref[1, pl.ds(0, N), :]
    h = agg * nsd_ref[pl.ds(0, N), 1:2]
    y = _united_norm_leaky(h, lam_ref[...], g_ref[...], b_ref[...])
    out_ref[...] = _leaky(jnp.mean(y, axis=0, keepdims=True))


_tc_norms = pl.pallas_call(
    _tc_norms_body,
    out_shape=jax.ShapeDtypeStruct((NP, 2), jnp.float32),
)

_tc_pre = pl.pallas_call(
    _tc_pre_body,
    out_shape=jax.ShapeDtypeStruct((NF, NP, FW), jnp.float32),
)

_tc_mid = pl.pallas_call(
    _tc_mid_body,
    out_shape=jax.ShapeDtypeStruct((NF, NP, FW), jnp.float32),
)

_tc_final = pl.pallas_call(
    _tc_final_body,
    out_shape=jax.ShapeDtypeStruct((1, D), jnp.float32),
)


def kernel(node_feats, edge_index, W1, W2, W3, lam1, lam2, lam3,
           gamma1, gamma2, gamma3, beta1, beta2, beta3):
    src = edge_index[0].astype(jnp.int32)
    dst = edge_index[1].astype(jnp.int32)
    pad = jnp.full((E_PAD - E,), N, jnp.int32)
    srcc = jnp.concatenate([src, pad]).reshape(NCHUNKS, CHUNK)
    dstc = jnp.concatenate([dst, pad]).reshape(NCHUNKS, CHUNK)

    deg_s, deg_d = _sc_deg(srcc, dstc)
    nsd = _tc_norms(deg_s, deg_d)
    h1 = _tc_pre(node_feats, W1, nsd)

    lams = [lam1.reshape(1, 3), lam2.reshape(1, 3), lam3.reshape(1, 3)]
    gammas = [gamma1.reshape(1, D), gamma2.reshape(1, D), gamma3.reshape(1, D)]
    betas = [beta1.reshape(1, D), beta2.reshape(1, D), beta3.reshape(1, D)]

    p1 = _sc_agg(h1, srcc, dstc)
    h2 = _tc_mid(p1, nsd, lams[0], gammas[0], betas[0], W2)
    p2 = _sc_agg(h2, srcc, dstc)
    h3 = _tc_mid(p2, nsd, lams[1], gammas[1], betas[1], W3)
    p3 = _sc_agg(h3, srcc, dstc)
    return _tc_final(p3, nsd, lams[2], gammas[2], betas[2])


# speedup vs baseline: 2.5390x; 2.5390x over previous
import jax
import jax.numpy as jnp
from jax import lax
from jax.experimental import pallas as pl
from jax.experimental.pallas import tpu as pltpu
from jax.experimental.pallas import tpu_sc as plsc

N = 10000
D = 128
E = 320000

NC = 2
NS = 16
LANES = 16

CHUNK = 128
NW = NC * NS
G = 4
CPT = ((E + CHUNK * NW * G - 1) // (CHUNK * NW * G)) * G
NCHUNKS = CPT * NW
E_PAD = NCHUNKS * CHUNK
NG = CPT // G
NG2 = NG // 2

NP = 10112
RPT = NP // NS
NF = 4
FW = D // NF
ZB = RPT // 2

_mesh = plsc.VectorSubcoreMesh(core_axis_name="c", subcore_axis_name="s")
_sc_params = pltpu.CompilerParams(use_tc_tiling_on_sc=False)


def _zero_rows(ref, nrows, width):
    z = jnp.zeros((LANES,), jnp.float32)

    def body(i, _):
        for t in range(width // LANES):
            ref[i, pl.ds(t * LANES, LANES)] = z
        return 0

    lax.fori_loop(0, nrows, body, 0, unroll=False)


def _sc_deg_body(srcc, dstc, out_s, out_d, sidx_all, didx_all, ones_v, stage,
                 sh_s, sh_d, sem_s):
    c = lax.axis_index("c")
    s = lax.axis_index("s")
    wid = c * NS + s

    pltpu.sync_copy(srcc.at[pl.ds(wid * CPT, CPT)], sidx_all)
    pltpu.sync_copy(dstc.at[pl.ds(wid * CPT, CPT)], didx_all)

    one = jnp.ones((LANES,), jnp.float32)

    def initones(i, _):
        ones_v[i, :] = one
        return 0

    lax.fori_loop(0, CHUNK, initones, 0, unroll=False)
    _zero_rows(stage, RPT, LANES)

    row0 = s * RPT
    pltpu.sync_copy(stage, sh_s.at[pl.ds(row0, RPT)])
    pltpu.sync_copy(stage, sh_d.at[pl.ds(row0, RPT)])
    plsc.subcore_barrier()

    def group_body(g, _):
        @pl.when(g > 0)
        def _():
            for _b in range(G):
                pltpu.make_async_copy(ones_v, sh_s.at[sidx_all.at[0]], sem_s).wait()
                pltpu.make_async_copy(ones_v, sh_d.at[didx_all.at[0]], sem_s).wait()

        for b in range(G):
            j = g * G + b
            pltpu.async_copy(ones_v, sh_s.at[sidx_all.at[j]], sem_s, add=True)
            pltpu.async_copy(ones_v, sh_d.at[didx_all.at[j]], sem_s, add=True)
        return 0

    lax.fori_loop(0, NG, group_body, 0, unroll=False)
    for _b in range(G):
        pltpu.make_async_copy(ones_v, sh_s.at[sidx_all.at[0]], sem_s).wait()
        pltpu.make_async_copy(ones_v, sh_d.at[didx_all.at[0]], sem_s).wait()
    plsc.subcore_barrier()

    pltpu.sync_copy(sh_s.at[pl.ds(row0, RPT)], stage)
    pltpu.sync_copy(stage, out_s.at[c, pl.ds(row0, RPT)])
    pltpu.sync_copy(sh_d.at[pl.ds(row0, RPT)], stage)
    pltpu.sync_copy(stage, out_d.at[c, pl.ds(row0, RPT)])


def _sc_agg_body(hg, srcc, dstc, out, sidx_all, didx_all, rows_g, zcbuf,
                 plane, sh_acc, sem_g, sem_s):
    c = lax.axis_index("c")
    s = lax.axis_index("s")
    wid = c * NS + s
    row0 = s * RPT

    pltpu.sync_copy(srcc.at[pl.ds(wid * CPT, CPT)], sidx_all)
    pltpu.sync_copy(dstc.at[pl.ds(wid * CPT, CPT)], didx_all)

    def _fire_g(p, grp, base):
        for b in range(G):
            pltpu.async_copy(
                plane.at[sidx_all.at[grp * G + b]], rows_g.at[base + b], sem_g)

    def _drain_g(p, grp, base):
        for b in range(G):
            pltpu.make_async_copy(
                plane.at[sidx_all.at[grp * G + b]], rows_g.at[base + b], sem_g).wait()

    def _fire_s(grp, base):
        for b in range(G):
            pltpu.async_copy(
                rows_g.at[base + b], sh_acc.at[didx_all.at[grp * G + b]], sem_s, add=True)

    def _drain_s(base):
        for b in range(G):
            pltpu.make_async_copy(
                rows_g.at[base + b], sh_acc.at[didx_all.at[0]], sem_s).wait()

    for p in range(NF):
        _zero_rows(zcbuf, ZB, FW)
        pltpu.sync_copy(zcbuf, sh_acc.at[pl.ds(row0, ZB)])
        pltpu.sync_copy(zcbuf, sh_acc.at[pl.ds(row0 + ZB, ZB)])
        pltpu.sync_copy(hg.at[p, pl.ds(row0, RPT)], plane.at[pl.ds(row0, RPT)])
        plsc.subcore_barrier()

        _fire_g(p, 0, 0)

        def pair_body(g2, _):
            @pl.when(g2 > 0)
            def _():
                _drain_s(G)
            _fire_g(p, 2 * g2 + 1, G)
            _drain_g(p, 2 * g2, 0)
            _fire_s(2 * g2, 0)
            _drain_g(p, 2 * g2 + 1, G)
            _fire_s(2 * g2 + 1, G)

            @pl.when(g2 < NG2 - 1)
            def _():
                _drain_s(0)
                _fire_g(p, 2 * g2 + 2, 0)
            return 0

        lax.fori_loop(0, NG2, pair_body, 0, unroll=False)
        _drain_s(0)
        _drain_s(G)
        plsc.subcore_barrier()

        for hb in range(2):
            r = row0 + hb * ZB
            pltpu.sync_copy(sh_acc.at[pl.ds(r, ZB)], zcbuf)
            pltpu.sync_copy(zcbuf, out.at[c, pl.ds(r, ZB), pl.ds(p * FW, FW)])


_sc_deg = jax.jit(pl.kernel(
    _sc_deg_body,
    out_type=(
        jax.ShapeDtypeStruct((NC, NP, LANES), jnp.float32),
        jax.ShapeDtypeStruct((NC, NP, LANES), jnp.float32),
    ),
    mesh=_mesh,
    compiler_params=_sc_params,
    scratch_types=[
        pltpu.VMEM((CPT, CHUNK), jnp.int32),
        pltpu.VMEM((CPT, CHUNK), jnp.int32),
        pltpu.VMEM((CHUNK, LANES), jnp.float32),
        pltpu.VMEM((RPT, LANES), jnp.float32),
        pltpu.VMEM_SHARED((NP, LANES), jnp.float32),
        pltpu.VMEM_SHARED((NP, LANES), jnp.float32),
        pltpu.SemaphoreType.DMA,
    ],
))

_sc_agg = jax.jit(pl.kernel(
    _sc_agg_body,
    out_type=jax.ShapeDtypeStruct((NC, NP, D), jnp.float32),
    mesh=_mesh,
    compiler_params=_sc_params,
    scratch_types=[
        pltpu.VMEM((CPT, CHUNK), jnp.int32),
        pltpu.VMEM((CPT, CHUNK), jnp.int32),
        pltpu.VMEM((2 * G, CHUNK, FW), jnp.float32),
        pltpu.VMEM((ZB, FW), jnp.float32),
        pltpu.VMEM_SHARED((NP, FW), jnp.float32),
        pltpu.VMEM_SHARED((NP, FW), jnp.float32),
        pltpu.SemaphoreType.DMA,
        pltpu.SemaphoreType.DMA,
    ],
))


def _leaky(x):
    return jnp.where(x >= 0, x, 0.2 * x)


def _tc_norms_body(ds_ref, dd_ref, nsd_out):
    deg_s = ds_ref[0, :, 0:1] + ds_ref[1, :, 0:1]
    deg_d = dd_ref[0, :, 0:1] + dd_ref[1, :, 0:1]
    nsd_out[:, 0:1] = jnp.where(deg_s > 0, lax.rsqrt(deg_s), 0.0)
    nsd_out[:, 1:2] = jnp.where(deg_d > 0, lax.rsqrt(deg_d), 0.0)


def _tc_pre_body(x_ref, w_ref, nsd_ref, h_out):
    h = jnp.dot(x_ref[...], w_ref[...], preferred_element_type=jnp.float32)
    hs = h * nsd_ref[pl.ds(0, N), 0:1]
    for p in range(NF):
        h_out[p, pl.ds(0, N), :] = hs[:, p * FW:(p + 1) * FW]
        h_out[p, pl.ds(N, NP - N), :] = jnp.zeros((NP - N, FW), jnp.float32)


def _united_norm_leaky(h, lam, gamma, beta):
    eps = 1e-5
    mn = jnp.mean(h, axis=1, keepdims=True)
    vn = jnp.mean((h - mn) ** 2, axis=1, keepdims=True)
    mb = jnp.mean(h, axis=0, keepdims=True)
    vb = jnp.mean((h - mb) ** 2, axis=0, keepdims=True)
    mg = jnp.mean(h)
    vg = jnp.mean((h - mg) ** 2)
    e = jnp.exp(lam - jnp.max(lam))
    sinv = 1.0 / jnp.sum(e)
    w0 = jnp.sum(e[:, 0:1]) * sinv
    w1 = jnp.sum(e[:, 1:2]) * sinv
    w2 = jnp.sum(e[:, 2:3]) * sinv
    rn = lax.rsqrt(vn + eps)
    rb = lax.rsqrt(vb + eps)
    rg = lax.rsqrt(vg + eps)
    scale = w0 * rn + w1 * rb + w2 * rg
    shift = w0 * mn * rn + w1 * mb * rb + w2 * mg * rg
    xh = h * scale - shift
    return _leaky(gamma * xh + beta)


def _tc_mid_body(p_ref, nsd_ref, lam_ref, g_ref, b_ref, w_ref, out_ref):
    agg = p_ref[0, pl.ds(0, N), :] + p_ref[1, pl.ds(0, N), :]
    h = agg * nsd_ref[pl.ds(0, N), 1:2]
    y = _united_norm_leaky(h, lam_ref[...], g_ref[...], b_ref[...])
    hn = jnp.dot(y, w_ref[...], preferred_element_type=jnp.float32)
    hs = hn * nsd_ref[pl.ds(0, N), 0:1]
    for p in range(NF):
        out_ref[p, pl.ds(0, N), :] = hs[:, p * FW:(p + 1) * FW]
        out_ref[p, pl.ds(N, NP - N), :] = jnp.zeros((NP - N, FW), jnp.float32)


def _tc_final_body(p_ref, nsd_ref, lam_ref, g_ref, b_ref, out_ref):
    agg = p_ref[0, pl.ds(0, N), :] + p_ref[1, pl.ds(0, N), :]
    h = agg * nsd_ref[pl.ds(0, N), 1:2]
    y = _united_norm_leaky(h, lam_ref[...], g_ref[...], b_ref[...])
    out_ref[...] = _leaky(jnp.mean(y, axis=0, keepdims=True))


_tc_norms = pl.pallas_call(
    _tc_norms_body,
    out_shape=jax.ShapeDtypeStruct((NP, 2), jnp.float32),
)

_tc_pre = pl.pallas_call(
    _tc_pre_body,
    out_shape=jax.ShapeDtypeStruct((NF, NP, FW), jnp.float32),
)

_tc_mid = pl.pallas_call(
    _tc_mid_body,
    out_shape=jax.ShapeDtypeStruct((NF, NP, FW), jnp.float32),
)

_tc_final = pl.pallas_call(
    _tc_final_body,
    out_shape=jax.ShapeDtypeStruct((1, D), jnp.float32),
)


def kernel(node_feats, edge_index, W1, W2, W3, lam1, lam2, lam3,
           gamma1, gamma2, gamma3, beta1, beta2, beta3):
    src = edge_index[0].astype(jnp.int32)
    dst = edge_index[1].astype(jnp.int32)
    pad = N + jnp.arange(E_PAD - E, dtype=jnp.int32) % (NP - N)
    srcc = jnp.concatenate([src, pad]).reshape(NCHUNKS, CHUNK)
    dstc = jnp.concatenate([dst, pad]).reshape(NCHUNKS, CHUNK)

    deg_s, deg_d = _sc_deg(srcc, dstc)
    nsd = _tc_norms(deg_s, deg_d)
    h1 = _tc_pre(node_feats, W1, nsd)

    lams = [lam1.reshape(1, 3), lam2.reshape(1, 3), lam3.reshape(1, 3)]
    gammas = [gamma1.reshape(1, D), gamma2.reshape(1, D), gamma3.reshape(1, D)]
    betas = [beta1.reshape(1, D), beta2.reshape(1, D), beta3.reshape(1, D)]

    p1 = _sc_agg(h1, srcc, dstc)
    h2 = _tc_mid(p1, nsd, lams[0], gammas[0], betas[0], W2)
    p2 = _sc_agg(h2, srcc, dstc)
    h3 = _tc_mid(p2, nsd, lams[1], gammas[1], betas[1], W3)
    p3 = _sc_agg(h3, srcc, dstc)
    return _tc_final(p3, nsd, lams[2], gammas[2], betas[2])


# speedup vs baseline: 2.6553x; 1.0458x over previous
import jax
import jax.numpy as jnp
from jax import lax
from jax.experimental import pallas as pl
from jax.experimental.pallas import tpu as pltpu
from jax.experimental.pallas import tpu_sc as plsc

N = 10000
D = 128
E = 320000

NC = 2
NS = 16
LANES = 16

CHUNK = 128
NW = NC * NS
G = 1
GD = 4
CPT = ((E + CHUNK * NW * 2 * G - 1) // (CHUNK * NW * 2 * G)) * 2 * G
NCHUNKS = CPT * NW
E_PAD = NCHUNKS * CHUNK
NGD = CPT // GD
NG2 = CPT // (2 * G)

NP = 10112
RPT = NP // NS
NF = 2
FW = D // NF
ZB = RPT // 4

_mesh = plsc.VectorSubcoreMesh(core_axis_name="c", subcore_axis_name="s")
_sc_params = pltpu.CompilerParams(use_tc_tiling_on_sc=False)


def _zero_rows(ref, nrows, width):
    z = jnp.zeros((LANES,), jnp.float32)

    def body(i, _):
        for t in range(width // LANES):
            ref[i, pl.ds(t * LANES, LANES)] = z
        return 0

    lax.fori_loop(0, nrows, body, 0, unroll=False)


def _sc_deg_body(srcc, dstc, out_s, out_d, sidx_all, didx_all, ones_v, stage,
                 sh_s, sh_d, sem_s):
    c = lax.axis_index("c")
    s = lax.axis_index("s")
    wid = c * NS + s

    pltpu.sync_copy(srcc.at[pl.ds(wid * CPT, CPT)], sidx_all)
    pltpu.sync_copy(dstc.at[pl.ds(wid * CPT, CPT)], didx_all)

    one = jnp.ones((LANES,), jnp.float32)

    def initones(i, _):
        ones_v[i, :] = one
        return 0

    lax.fori_loop(0, CHUNK, initones, 0, unroll=False)
    _zero_rows(stage, RPT, LANES)

    row0 = s * RPT
    pltpu.sync_copy(stage, sh_s.at[pl.ds(row0, RPT)])
    pltpu.sync_copy(stage, sh_d.at[pl.ds(row0, RPT)])
    plsc.subcore_barrier()

    def group_body(g, _):
        @pl.when(g > 0)
        def _():
            for _b in range(GD):
                pltpu.make_async_copy(ones_v, sh_s.at[sidx_all.at[0]], sem_s).wait()
                pltpu.make_async_copy(ones_v, sh_d.at[didx_all.at[0]], sem_s).wait()

        for b in range(GD):
            j = g * GD + b
            pltpu.async_copy(ones_v, sh_s.at[sidx_all.at[j]], sem_s, add=True)
            pltpu.async_copy(ones_v, sh_d.at[didx_all.at[j]], sem_s, add=True)
        return 0

    lax.fori_loop(0, NGD, group_body, 0, unroll=False)
    for _b in range(GD):
        pltpu.make_async_copy(ones_v, sh_s.at[sidx_all.at[0]], sem_s).wait()
        pltpu.make_async_copy(ones_v, sh_d.at[didx_all.at[0]], sem_s).wait()
    plsc.subcore_barrier()

    pltpu.sync_copy(sh_s.at[pl.ds(row0, RPT)], stage)
    pltpu.sync_copy(stage, out_s.at[c, pl.ds(row0, RPT)])
    pltpu.sync_copy(sh_d.at[pl.ds(row0, RPT)], stage)
    pltpu.sync_copy(stage, out_d.at[c, pl.ds(row0, RPT)])


def _sc_agg_body(hg, srcc, dstc, out, sidx_all, didx_all, rows_g, zcbuf,
                 plane, sh_acc, sem_g, sem_s):
    c = lax.axis_index("c")
    s = lax.axis_index("s")
    wid = c * NS + s
    row0 = s * RPT

    pltpu.sync_copy(srcc.at[pl.ds(wid * CPT, CPT)], sidx_all)
    pltpu.sync_copy(dstc.at[pl.ds(wid * CPT, CPT)], didx_all)

    def _fire_g(p, grp, base):
        for b in range(G):
            pltpu.async_copy(
                plane.at[sidx_all.at[grp * G + b]], rows_g.at[base + b], sem_g)

    def _drain_g(p, grp, base):
        for b in range(G):
            pltpu.make_async_copy(
                plane.at[sidx_all.at[grp * G + b]], rows_g.at[base + b], sem_g).wait()

    def _fire_s(grp, base):
        for b in range(G):
            pltpu.async_copy(
                rows_g.at[base + b], sh_acc.at[didx_all.at[grp * G + b]], sem_s, add=True)

    def _drain_s(base):
        for b in range(G):
            pltpu.make_async_copy(
                rows_g.at[base + b], sh_acc.at[didx_all.at[0]], sem_s).wait()

    for p in range(NF):
        _zero_rows(zcbuf, ZB, FW)
        for zb in range(RPT // ZB):
            pltpu.sync_copy(zcbuf, sh_acc.at[pl.ds(row0 + zb * ZB, ZB)])
        pltpu.sync_copy(hg.at[p, pl.ds(row0, RPT)], plane.at[pl.ds(row0, RPT)])
        plsc.subcore_barrier()

        _fire_g(p, 0, 0)

        def pair_body(g2, _):
            @pl.when(g2 > 0)
            def _():
                _drain_s(G)
            _fire_g(p, 2 * g2 + 1, G)
            _drain_g(p, 2 * g2, 0)
            _fire_s(2 * g2, 0)
            _drain_g(p, 2 * g2 + 1, G)
            _fire_s(2 * g2 + 1, G)

            @pl.when(g2 < NG2 - 1)
            def _():
                _drain_s(0)
                _fire_g(p, 2 * g2 + 2, 0)
            return 0

        lax.fori_loop(0, NG2, pair_body, 0, unroll=False)
        _drain_s(0)
        _drain_s(G)
        plsc.subcore_barrier()

        for hb in range(RPT // ZB):
            r = row0 + hb * ZB
            pltpu.sync_copy(sh_acc.at[pl.ds(r, ZB)], zcbuf)
            pltpu.sync_copy(zcbuf, out.at[c, pl.ds(r, ZB), pl.ds(p * FW, FW)])


_sc_deg = jax.jit(pl.kernel(
    _sc_deg_body,
    out_type=(
        jax.ShapeDtypeStruct((NC, NP, LANES), jnp.float32),
        jax.ShapeDtypeStruct((NC, NP, LANES), jnp.float32),
    ),
    mesh=_mesh,
    compiler_params=_sc_params,
    scratch_types=[
        pltpu.VMEM((CPT, CHUNK), jnp.int32),
        pltpu.VMEM((CPT, CHUNK), jnp.int32),
        pltpu.VMEM((CHUNK, LANES), jnp.float32),
        pltpu.VMEM((RPT, LANES), jnp.float32),
        pltpu.VMEM_SHARED((NP, LANES), jnp.float32),
        pltpu.VMEM_SHARED((NP, LANES), jnp.float32),
        pltpu.SemaphoreType.DMA,
    ],
))

_sc_agg = jax.jit(pl.kernel(
    _sc_agg_body,
    out_type=jax.ShapeDtypeStruct((NC, NP, D), jnp.float32),
    mesh=_mesh,
    compiler_params=_sc_params,
    scratch_types=[
        pltpu.VMEM((CPT, CHUNK), jnp.int32),
        pltpu.VMEM((CPT, CHUNK), jnp.int32),
        pltpu.VMEM((2 * G, CHUNK, FW), jnp.float32),
        pltpu.VMEM((ZB, FW), jnp.float32),
        pltpu.VMEM_SHARED((NP, FW), jnp.float32),
        pltpu.VMEM_SHARED((NP, FW), jnp.float32),
        pltpu.SemaphoreType.DMA,
        pltpu.SemaphoreType.DMA,
    ],
))


def _leaky(x):
    return jnp.where(x >= 0, x, 0.2 * x)


def _tc_norms_body(ds_ref, dd_ref, nsd_out):
    deg_s = ds_ref[0, :, 0:1] + ds_ref[1, :, 0:1]
    deg_d = dd_ref[0, :, 0:1] + dd_ref[1, :, 0:1]
    nsd_out[:, 0:1] = jnp.where(deg_s > 0, lax.rsqrt(deg_s), 0.0)
    nsd_out[:, 1:2] = jnp.where(deg_d > 0, lax.rsqrt(deg_d), 0.0)


def _tc_pre_body(x_ref, w_ref, nsd_ref, h_out):
    h = jnp.dot(x_ref[...], w_ref[...], preferred_element_type=jnp.float32)
    hs = h * nsd_ref[pl.ds(0, N), 0:1]
    for p in range(NF):
        h_out[p, pl.ds(0, N), :] = hs[:, p * FW:(p + 1) * FW]
        h_out[p, pl.ds(N, NP - N), :] = jnp.zeros((NP - N, FW), jnp.float32)


def _united_norm_leaky(h, lam, gamma, beta):
    eps = 1e-5
    mn = jnp.mean(h, axis=1, keepdims=True)
    vn = jnp.mean((h - mn) ** 2, axis=1, keepdims=True)
    mb = jnp.mean(h, axis=0, keepdims=True)
    vb = jnp.mean((h - mb) ** 2, axis=0, keepdims=True)
    mg = jnp.mean(h)
    vg = jnp.mean((h - mg) ** 2)
    e = jnp.exp(lam - jnp.max(lam))
    sinv = 1.0 / jnp.sum(e)
    w0 = jnp.sum(e[:, 0:1]) * sinv
    w1 = jnp.sum(e[:, 1:2]) * sinv
    w2 = jnp.sum(e[:, 2:3]) * sinv
    rn = lax.rsqrt(vn + eps)
    rb = lax.rsqrt(vb + eps)
    rg = lax.rsqrt(vg + eps)
    scale = w0 * rn + w1 * rb + w2 * rg
    shift = w0 * mn * rn + w1 * mb * rb + w2 * mg * rg
    xh = h * scale - shift
    return _leaky(gamma * xh + beta)


def _tc_mid_body(p_ref, nsd_ref, lam_ref, g_ref, b_ref, w_ref, out_ref):
    agg = p_ref[0, pl.ds(0, N), :] + p_ref[1, pl.ds(0, N), :]
    h = agg * nsd_ref[pl.ds(0, N), 1:2]
    y = _united_norm_leaky(h, lam_ref[...], g_ref[...], b_ref[...])
    hn = jnp.dot(y, w_ref[...], preferred_element_type=jnp.float32)
    hs = hn * nsd_ref[pl.ds(0, N), 0:1]
    for p in range(NF):
        out_ref[p, pl.ds(0, N), :] = hs[:, p * FW:(p + 1) * FW]
        out_ref[p, pl.ds(N, NP - N), :] = jnp.zeros((NP - N, FW), jnp.float32)


def _tc_final_body(p_ref, nsd_ref, lam_ref, g_ref, b_ref, out_ref):
    agg = p_ref[0, pl.ds(0, N), :] + p_ref[1, pl.ds(0, N), :]
    h = agg * nsd_ref[pl.ds(0, N), 1:2]
    y = _united_norm_leaky(h, lam_ref[...], g_ref[...], b_ref[...])
    out_ref[...] = _leaky(jnp.mean(y, axis=0, keepdims=True))


_tc_norms = pl.pallas_call(
    _tc_norms_body,
    out_shape=jax.ShapeDtypeStruct((NP, 2), jnp.float32),
)

_tc_pre = pl.pallas_call(
    _tc_pre_body,
    out_shape=jax.ShapeDtypeStruct((NF, NP, FW), jnp.float32),
)

_tc_mid = pl.pallas_call(
    _tc_mid_body,
    out_shape=jax.ShapeDtypeStruct((NF, NP, FW), jnp.float32),
)

_tc_final = pl.pallas_call(
    _tc_final_body,
    out_shape=jax.ShapeDtypeStruct((1, D), jnp.float32),
)


def kernel(node_feats, edge_index, W1, W2, W3, lam1, lam2, lam3,
           gamma1, gamma2, gamma3, beta1, beta2, beta3):
    src = edge_index[0].astype(jnp.int32)
    dst = edge_index[1].astype(jnp.int32)
    pad = N + jnp.arange(E_PAD - E, dtype=jnp.int32) % (NP - N)
    srcc = jnp.concatenate([src, pad]).reshape(NCHUNKS, CHUNK)
    dstc = jnp.concatenate([dst, pad]).reshape(NCHUNKS, CHUNK)

    deg_s, deg_d = _sc_deg(srcc, dstc)
    nsd = _tc_norms(deg_s, deg_d)
    h1 = _tc_pre(node_feats, W1, nsd)

    lams = [lam1.reshape(1, 3), lam2.reshape(1, 3), lam3.reshape(1, 3)]
    gammas = [gamma1.reshape(1, D), gamma2.reshape(1, D), gamma3.reshape(1, D)]
    betas = [beta1.reshape(1, D), beta2.reshape(1, D), beta3.reshape(1, D)]

    p1 = _sc_agg(h1, srcc, dstc)
    h2 = _tc_mid(p1, nsd, lams[0], gammas[0], betas[0], W2)
    p2 = _sc_agg(h2, srcc, dstc)
    h3 = _tc_mid(p2, nsd, lams[1], gammas[1], betas[1], W3)
    p3 = _sc_agg(h3, srcc, dstc)
    return _tc_final(p3, nsd, lams[2], gammas[2], betas[2])


# speedup vs baseline: 2.6744x; 1.0072x over previous
import jax
import jax.numpy as jnp
from jax import lax
from jax.experimental import pallas as pl
from jax.experimental.pallas import tpu as pltpu
from jax.experimental.pallas import tpu_sc as plsc

N = 10000
D = 128
E = 320000

NC = 2
NS = 16
LANES = 16

CHUNK = 128
NW = NC * NS
G = 1
GD = 4
CPT = ((E + CHUNK * NW * 2 * G - 1) // (CHUNK * NW * 2 * G)) * 2 * G
NCHUNKS = CPT * NW
E_PAD = NCHUNKS * CHUNK
NGD = CPT // GD
NG2 = CPT // (2 * G)

NP = 10112
RPT = NP // NS
NF = 2
FW = D // NF
ZB = RPT // 4

_mesh = plsc.VectorSubcoreMesh(core_axis_name="c", subcore_axis_name="s")
_sc_params = pltpu.CompilerParams(use_tc_tiling_on_sc=False)


def _zero_rows(ref, nrows, width):
    z = jnp.zeros((LANES,), jnp.float32)

    def body(i, _):
        for t in range(width // LANES):
            ref[i, pl.ds(t * LANES, LANES)] = z
        return 0

    lax.fori_loop(0, nrows, body, 0, unroll=False)


def _sc_deg_body(srcc, dstc, out_s, out_d, sidx_all, didx_all, ones_v, stage,
                 sh_s, sh_d, sem_s):
    c = lax.axis_index("c")
    s = lax.axis_index("s")
    wid = c * NS + s

    pltpu.sync_copy(srcc.at[pl.ds(wid * CPT, CPT)], sidx_all)
    pltpu.sync_copy(dstc.at[pl.ds(wid * CPT, CPT)], didx_all)

    one = jnp.ones((LANES,), jnp.float32)

    def initones(i, _):
        ones_v[i, :] = one
        return 0

    lax.fori_loop(0, CHUNK, initones, 0, unroll=False)
    _zero_rows(stage, RPT, LANES)

    row0 = s * RPT
    pltpu.sync_copy(stage, sh_s.at[pl.ds(row0, RPT)])
    pltpu.sync_copy(stage, sh_d.at[pl.ds(row0, RPT)])
    plsc.subcore_barrier()

    def group_body(g, _):
        @pl.when(g > 0)
        def _():
            for _b in range(GD):
                pltpu.make_async_copy(ones_v, sh_s.at[sidx_all.at[0]], sem_s).wait()
                pltpu.make_async_copy(ones_v, sh_d.at[didx_all.at[0]], sem_s).wait()

        for b in range(GD):
            j = g * GD + b
            pltpu.async_copy(ones_v, sh_s.at[sidx_all.at[j]], sem_s, add=True)
            pltpu.async_copy(ones_v, sh_d.at[didx_all.at[j]], sem_s, add=True)
        return 0

    lax.fori_loop(0, NGD, group_body, 0, unroll=False)
    for _b in range(GD):
        pltpu.make_async_copy(ones_v, sh_s.at[sidx_all.at[0]], sem_s).wait()
        pltpu.make_async_copy(ones_v, sh_d.at[didx_all.at[0]], sem_s).wait()
    plsc.subcore_barrier()

    pltpu.sync_copy(sh_s.at[pl.ds(row0, RPT)], stage)
    pltpu.sync_copy(stage, out_s.at[c, pl.ds(row0, RPT)])
    pltpu.sync_copy(sh_d.at[pl.ds(row0, RPT)], stage)
    pltpu.sync_copy(stage, out_d.at[c, pl.ds(row0, RPT)])


def _sc_agg_body(hg, srcc, dstc, out, sidx_all, didx_all, rows_g, zcbuf,
                 plane, sh_acc, sem_g, sem_s):
    c = lax.axis_index("c")
    s = lax.axis_index("s")
    wid = c * NS + s
    row0 = s * RPT

    pltpu.sync_copy(srcc.at[pl.ds(wid * CPT, CPT)], sidx_all)
    pltpu.sync_copy(dstc.at[pl.ds(wid * CPT, CPT)], didx_all)

    def _fire_g(p, grp, base):
        for b in range(G):
            pltpu.async_copy(
                plane.at[sidx_all.at[grp * G + b]], rows_g.at[base + b], sem_g)

    def _drain_g(p, grp, base):
        for b in range(G):
            pltpu.make_async_copy(
                plane.at[sidx_all.at[grp * G + b]], rows_g.at[base + b], sem_g).wait()

    def _fire_s(grp, base):
        for b in range(G):
            pltpu.async_copy(
                rows_g.at[base + b], sh_acc.at[didx_all.at[grp * G + b]], sem_s, add=True)

    def _drain_s(base):
        for b in range(G):
            pltpu.make_async_copy(
                rows_g.at[base + b], sh_acc.at[didx_all.at[0]], sem_s).wait()

    for p in range(NF):
        _zero_rows(zcbuf, ZB, FW)
        for zb in range(RPT // ZB):
            pltpu.sync_copy(zcbuf, sh_acc.at[pl.ds(row0 + zb * ZB, ZB)])
        pltpu.sync_copy(hg.at[p, pl.ds(row0, RPT)], plane.at[pl.ds(row0, RPT)])
        plsc.subcore_barrier()

        _fire_g(p, 0, 0)

        def pair_body(g2, _):
            @pl.when(g2 > 0)
            def _():
                _drain_s(G)
            _fire_g(p, 2 * g2 + 1, G)
            _drain_g(p, 2 * g2, 0)
            _fire_s(2 * g2, 0)
            _drain_g(p, 2 * g2 + 1, G)
            _fire_s(2 * g2 + 1, G)

            @pl.when(g2 < NG2 - 1)
            def _():
                _drain_s(0)
                _fire_g(p, 2 * g2 + 2, 0)
            return 0

        lax.fori_loop(0, NG2, pair_body, 0, unroll=False)
        _drain_s(0)
        _drain_s(G)
        plsc.subcore_barrier()

        for hb in range(RPT // ZB):
            r = row0 + hb * ZB
            pltpu.sync_copy(sh_acc.at[pl.ds(r, ZB)], zcbuf)
            pltpu.sync_copy(zcbuf, out.at[c, pl.ds(r, ZB), pl.ds(p * FW, FW)])


_sc_deg = jax.jit(pl.kernel(
    _sc_deg_body,
    out_type=(
        jax.ShapeDtypeStruct((NC, NP, LANES), jnp.float32),
        jax.ShapeDtypeStruct((NC, NP, LANES), jnp.float32),
    ),
    mesh=_mesh,
    compiler_params=_sc_params,
    scratch_types=[
        pltpu.VMEM((CPT, CHUNK), jnp.int32),
        pltpu.VMEM((CPT, CHUNK), jnp.int32),
        pltpu.VMEM((CHUNK, LANES), jnp.float32),
        pltpu.VMEM((RPT, LANES), jnp.float32),
        pltpu.VMEM_SHARED((NP, LANES), jnp.float32),
        pltpu.VMEM_SHARED((NP, LANES), jnp.float32),
        pltpu.SemaphoreType.DMA,
    ],
))

_sc_agg = jax.jit(pl.kernel(
    _sc_agg_body,
    out_type=jax.ShapeDtypeStruct((NC, NP, D), jnp.float32),
    mesh=_mesh,
    compiler_params=_sc_params,
    scratch_types=[
        pltpu.VMEM((CPT, CHUNK), jnp.int32),
        pltpu.VMEM((CPT, CHUNK), jnp.int32),
        pltpu.VMEM((2 * G, CHUNK, FW), jnp.float32),
        pltpu.VMEM((ZB, FW), jnp.float32),
        pltpu.VMEM_SHARED((NP, FW), jnp.float32),
        pltpu.VMEM_SHARED((NP, FW), jnp.float32),
        pltpu.SemaphoreType.DMA,
        pltpu.SemaphoreType.DMA,
    ],
))


def _leaky(x):
    return jnp.where(x >= 0, x, 0.2 * x)


def _tc_pre_body(x_ref, w_ref, ds_ref, dd_ref, h_out, nsd_out):
    deg_s = ds_ref[0, :, 0:1] + ds_ref[1, :, 0:1]
    deg_d = dd_ref[0, :, 0:1] + dd_ref[1, :, 0:1]
    ns = jnp.where(deg_s > 0, lax.rsqrt(deg_s), 0.0)
    nsd_out[:, 0:1] = ns
    nsd_out[:, 1:2] = jnp.where(deg_d > 0, lax.rsqrt(deg_d), 0.0)
    h = jnp.dot(x_ref[...], w_ref[...], preferred_element_type=jnp.float32)
    hs = h * ns[:N, :]
    for p in range(NF):
        h_out[p, pl.ds(0, N), :] = hs[:, p * FW:(p + 1) * FW]
        h_out[p, pl.ds(N, NP - N), :] = jnp.zeros((NP - N, FW), jnp.float32)


def _united_norm_leaky(h, lam, gamma, beta):
    eps = 1e-5
    mn = jnp.mean(h, axis=1, keepdims=True)
    vn = jnp.mean((h - mn) ** 2, axis=1, keepdims=True)
    mb = jnp.mean(h, axis=0, keepdims=True)
    vb = jnp.mean((h - mb) ** 2, axis=0, keepdims=True)
    mg = jnp.mean(h)
    vg = jnp.mean((h - mg) ** 2)
    e = jnp.exp(lam - jnp.max(lam))
    sinv = 1.0 / jnp.sum(e)
    w0 = jnp.sum(e[:, 0:1]) * sinv
    w1 = jnp.sum(e[:, 1:2]) * sinv
    w2 = jnp.sum(e[:, 2:3]) * sinv
    rn = lax.rsqrt(vn + eps)
    rb = lax.rsqrt(vb + eps)
    rg = lax.rsqrt(vg + eps)
    scale = w0 * rn + w1 * rb + w2 * rg
    shift = w0 * mn * rn + w1 * mb * rb + w2 * mg * rg
    xh = h * scale - shift
    return _leaky(gamma * xh + beta)


def _tc_mid_body(p_ref, nsd_ref, lam_ref, g_ref, b_ref, w_ref, out_ref):
    agg = p_ref[0, pl.ds(0, N), :] + p_ref[1, pl.ds(0, N), :]
    h = agg * nsd_ref[pl.ds(0, N), 1:2]
    y = _united_norm_leaky(h, lam_ref[...], g_ref[...], b_ref[...])
    hn = jnp.dot(y, w_ref[...], preferred_element_type=jnp.float32)
    hs = hn * nsd_ref[pl.ds(0, N), 0:1]
    for p in range(NF):
        out_ref[p, pl.ds(0, N), :] = hs[:, p * FW:(p + 1) * FW]
        out_ref[p, pl.ds(N, NP - N), :] = jnp.zeros((NP - N, FW), jnp.float32)


def _tc_final_body(p_ref, nsd_ref, lam_ref, g_ref, b_ref, out_ref):
    agg = p_ref[0, pl.ds(0, N), :] + p_ref[1, pl.ds(0, N), :]
    h = agg * nsd_ref[pl.ds(0, N), 1:2]
    y = _united_norm_leaky(h, lam_ref[...], g_ref[...], b_ref[...])
    out_ref[...] = _leaky(jnp.mean(y, axis=0, keepdims=True))


_tc_pre = pl.pallas_call(
    _tc_pre_body,
    out_shape=(
        jax.ShapeDtypeStruct((NF, NP, FW), jnp.float32),
        jax.ShapeDtypeStruct((NP, 2), jnp.float32),
    ),
)

_tc_mid = pl.pallas_call(
    _tc_mid_body,
    out_shape=jax.ShapeDtypeStruct((NF, NP, FW), jnp.float32),
)

_tc_final = pl.pallas_call(
    _tc_final_body,
    out_shape=jax.ShapeDtypeStruct((1, D), jnp.float32),
)


def kernel(node_feats, edge_index, W1, W2, W3, lam1, lam2, lam3,
           gamma1, gamma2, gamma3, beta1, beta2, beta3):
    src = edge_index[0].astype(jnp.int32)
    dst = edge_index[1].astype(jnp.int32)
    pad = N + jnp.arange(E_PAD - E, dtype=jnp.int32) % (NP - N)
    srcc = jnp.concatenate([src, pad]).reshape(NCHUNKS, CHUNK)
    dstc = jnp.concatenate([dst, pad]).reshape(NCHUNKS, CHUNK)

    deg_s, deg_d = _sc_deg(srcc, dstc)
    h1, nsd = _tc_pre(node_feats, W1, deg_s, deg_d)

    lams = [lam1.reshape(1, 3), lam2.reshape(1, 3), lam3.reshape(1, 3)]
    gammas = [gamma1.reshape(1, D), gamma2.reshape(1, D), gamma3.reshape(1, D)]
    betas = [beta1.reshape(1, D), beta2.reshape(1, D), beta3.reshape(1, D)]

    p1 = _sc_agg(h1, srcc, dstc)
    h2 = _tc_mid(p1, nsd, lams[0], gammas[0], betas[0], W2)
    p2 = _sc_agg(h2, srcc, dstc)
    h3 = _tc_mid(p2, nsd, lams[1], gammas[1], betas[1], W3)
    p3 = _sc_agg(h3, srcc, dstc)
    return _tc_final(p3, nsd, lams[2], gammas[2], betas[2])
